# baseline (speedup 1.0000x reference)
import jax
import jax.numpy as jnp
from jax import lax
from jax.experimental import pallas as pl
from jax.experimental.pallas import tpu as pltpu
from jax.experimental.pallas import tpu_sc as plsc

NC = 2
NS = 16
L = 16
NW = NC * NS
B = 16384
D = 32
PK = 128 // D
BPW = B // NW
CT = 128
NCH = BPW // CT


def _dot_body(users_hbm, items_hbm, uemb_hbm, iemb_hbm, out_hbm,
              uidx_v, iidx_v, utid_v, itid_v, ubuf_v, ibuf_v,
              hbuf_v, out_v, sem_u, sem_i):
    wid = lax.axis_index("s") * NC + lax.axis_index("c")
    base = wid * BPW

    pltpu.sync_copy(users_hbm.at[pl.ds(base, BPW)], uidx_v)
    pltpu.sync_copy(items_hbm.at[pl.ds(base, BPW)], iidx_v)

    def tid(j, carry):
        sl = pl.ds(j * L, L)
        utid_v[sl] = lax.shift_right_logical(uidx_v[sl], 2)
        itid_v[sl] = lax.shift_right_logical(iidx_v[sl], 2)
        return carry

    lax.fori_loop(0, BPW // L, tid, 0)

    lanes = lax.iota(jnp.int32, L)
    scatter_base = lanes * CT

    def chunk(c, carry):
        sl = pl.ds(c * CT, CT)
        pltpu.async_copy(uemb_hbm.at[utid_v.at[sl]], ubuf_v, sem_u)
        pltpu.async_copy(iemb_hbm.at[itid_v.at[sl]], ibuf_v, sem_i)
        pltpu.make_async_copy(uemb_hbm.at[pl.ds(0, CT)], ubuf_v, sem_u).wait()
        pltpu.make_async_copy(iemb_hbm.at[pl.ds(0, CT)], ibuf_v, sem_i).wait()

        def row_pass(j0, carry2):
            uvec = uidx_v[pl.ds(c * CT + j0 * L, L)]
            ivec = iidx_v[pl.ds(c * CT + j0 * L, L)]
            ucols = lax.shift_left(lax.bitwise_and(uvec, PK - 1), 5)
            icols = lax.shift_left(lax.bitwise_and(ivec, PK - 1), 5)
            for k in range(L):
                j = j0 * L + k
                uc = ucols[k]
                ic = icols[k]
                u0 = ubuf_v[j, pl.ds(uc, L)]
                u1 = ubuf_v[j, pl.ds(uc + L, L)]
                i0 = ibuf_v[j, pl.ds(ic, L)]
                i1 = ibuf_v[j, pl.ds(ic + L, L)]
                h = u0 * i0 + u1 * i1
                plsc.store_scatter(hbuf_v, [scatter_base + j], h)
            return carry2

        lax.fori_loop(0, CT // L, row_pass, 0)

        def group_pass(g, carry2):
            acc = jnp.zeros((L,), jnp.float32)
            for d in range(L):
                acc = acc + hbuf_v[pl.ds(d * CT + g * L, L)]
            out_v[pl.ds(c * CT + g * L, L)] = acc
            return carry2

        lax.fori_loop(0, CT // L, group_pass, 0)
        return carry

    lax.fori_loop(0, NCH, chunk, 0)

    pltpu.sync_copy(out_v, out_hbm.at[pl.ds(base, BPW)])


def kernel(users, items, user_emb, item_emb):
    upk = user_emb.reshape(user_emb.shape[0] // PK, PK * D)
    ipk = item_emb.reshape(item_emb.shape[0] // PK, PK * D)
    mesh = plsc.VectorSubcoreMesh(core_axis_name="c", subcore_axis_name="s")
    run = pl.kernel(
        _dot_body,
        out_type=jax.ShapeDtypeStruct((B,), jnp.float32),
        mesh=mesh,
        compiler_params=pltpu.CompilerParams(
            needs_layout_passes=False, use_tc_tiling_on_sc=True),
        scratch_types=[
            pltpu.VMEM((BPW,), jnp.int32),
            pltpu.VMEM((BPW,), jnp.int32),
            pltpu.VMEM((BPW,), jnp.int32),
            pltpu.VMEM((BPW,), jnp.int32),
            pltpu.VMEM((CT, PK * D), jnp.float32),
            pltpu.VMEM((CT, PK * D), jnp.float32),
            pltpu.VMEM((L * CT,), jnp.float32),
            pltpu.VMEM((BPW,), jnp.float32),
            pltpu.SemaphoreType.DMA,
            pltpu.SemaphoreType.DMA,
        ],
    )
    return run(users.astype(jnp.int32), items.astype(jnp.int32), upk, ipk)
